# Initial kernel scaffold; baseline (speedup 1.0000x reference)
import jax
import jax.numpy as jnp
from jax import lax
from jax.experimental import pallas as pl
from jax.experimental.pallas import tpu as pltpu
from jax.experimental.pallas import tpu_sc as plsc

N = 10000
E = 320000
D = 128
BLK = 128
NB = E // BLK
NC = 2
NS = 16
NW = NC * NS
ROWS_PER_TILE = N // NS
MAX_ITERS = (NB + NW - 1) // NW

_MESH = plsc.VectorSubcoreMesh(
    core_axis_name="c", subcore_axis_name="s", num_cores=NC, num_subcores=NS
)


def _hist_body(s2d_hbm, ones_hbm, zeros_hbm, out_hbm, sidx_v, ones_v, dacc):
    c = lax.axis_index("c")
    s = lax.axis_index("s")
    wid = s * NC + c
    r0 = s * ROWS_PER_TILE
    pltpu.sync_copy(zeros_hbm.at[pl.ds(r0, ROWS_PER_TILE)],
                    dacc.at[pl.ds(r0, ROWS_PER_TILE)])
    pltpu.sync_copy(ones_hbm, ones_v)
    plsc.subcore_barrier()

    def body(k, carry):
        j = wid + NW * k

        @pl.when(j < NB)
        def _():
            pltpu.sync_copy(s2d_hbm.at[j], sidx_v)
            pltpu.sync_copy(ones_v, dacc.at[sidx_v], add=True)

        return carry

    lax.fori_loop(0, MAX_ITERS, body, 0)
    plsc.subcore_barrier()
    pltpu.sync_copy(dacc.at[pl.ds(r0, ROWS_PER_TILE)],
                    out_hbm.at[c, pl.ds(r0, ROWS_PER_TILE)])


_hist = pl.kernel(
    _hist_body,
    out_type=jax.ShapeDtypeStruct((NC, N, 8), jnp.float32),
    mesh=_MESH,
    scratch_types=[
        pltpu.VMEM((BLK,), jnp.int32),
        pltpu.VMEM((BLK, 8), jnp.float32),
        pltpu.VMEM_SHARED((N, 8), jnp.float32),
    ],
)


def _agg_body(h_hbm, s2d_hbm, r2d_hbm, zeros_hbm, out_hbm,
              sidx_v, ridx_v, rows_v, sem, acc):
    c = lax.axis_index("c")
    s = lax.axis_index("s")
    wid = s * NC + c
    r0 = s * ROWS_PER_TILE

    @pl.when(c == 0)
    def _():
        pltpu.sync_copy(h_hbm.at[pl.ds(r0, ROWS_PER_TILE)],
                        acc.at[pl.ds(r0, ROWS_PER_TILE)])

    @pl.when(c == 1)
    def _():
        pltpu.sync_copy(zeros_hbm.at[pl.ds(r0, ROWS_PER_TILE)],
                        acc.at[pl.ds(r0, ROWS_PER_TILE)])

    plsc.subcore_barrier()

    def body(k, carry):
        j = wid + NW * k

        @pl.when(j < NB)
        def _():
            pltpu.sync_copy(s2d_hbm.at[j], sidx_v)
            pltpu.sync_copy(r2d_hbm.at[j], ridx_v)
            pltpu.async_copy(h_hbm.at[sidx_v], rows_v, sem).wait()
            pltpu.sync_copy(rows_v, acc.at[ridx_v], add=True)

        return carry

    lax.fori_loop(0, MAX_ITERS, body, 0)
    plsc.subcore_barrier()
    pltpu.sync_copy(acc.at[pl.ds(r0, ROWS_PER_TILE)],
                    out_hbm.at[c, pl.ds(r0, ROWS_PER_TILE)])


_agg = pl.kernel(
    _agg_body,
    out_type=jax.ShapeDtypeStruct((NC, N, D), jnp.float32),
    mesh=_MESH,
    scratch_types=[
        pltpu.VMEM((BLK,), jnp.int32),
        pltpu.VMEM((BLK,), jnp.int32),
        pltpu.VMEM((BLK, D), jnp.float32),
        pltpu.SemaphoreType.DMA,
        pltpu.VMEM_SHARED((N, D), jnp.float32),
    ],
)

_MM_ROWS = 1000


def _mm_body(nodes_ref, w_ref, b_ref, degp_ref, out_ref):
    deg = degp_ref[0, :, 0] + degp_ref[1, :, 0] + 1.0
    h = jnp.dot(nodes_ref[...], w_ref[...], preferred_element_type=jnp.float32)
    h = jnp.maximum(h + b_ref[...], 0.0)
    out_ref[...] = h * (1.0 / deg)[:, None]


def _matmul_scaled(nodes, w, b2d, degp):
    grid = N // _MM_ROWS
    return pl.pallas_call(
        _mm_body,
        grid=(grid,),
        in_specs=[
            pl.BlockSpec((_MM_ROWS, D), lambda i: (i, 0)),
            pl.BlockSpec((D, D), lambda i: (0, 0)),
            pl.BlockSpec((1, D), lambda i: (0, 0)),
            pl.BlockSpec((NC, _MM_ROWS, 8), lambda i: (0, i, 0)),
        ],
        out_specs=pl.BlockSpec((_MM_ROWS, D), lambda i: (i, 0)),
        out_shape=jax.ShapeDtypeStruct((N, D), jnp.float32),
    )(nodes, w, b2d, degp)


def _add_body(p_ref, out_ref):
    out_ref[...] = p_ref[0] + p_ref[1]


def _combine(parts):
    grid = N // _MM_ROWS
    return pl.pallas_call(
        _add_body,
        grid=(grid,),
        in_specs=[pl.BlockSpec((NC, _MM_ROWS, D), lambda i: (0, i, 0))],
        out_specs=pl.BlockSpec((_MM_ROWS, D), lambda i: (i, 0)),
        out_shape=jax.ShapeDtypeStruct((N, D), jnp.float32),
    )(parts)


def kernel(nodes, senders, receivers, W, b):
    s2d = senders.reshape(NB, BLK)
    r2d = receivers.reshape(NB, BLK)
    ones8 = jnp.ones((BLK, 8), jnp.float32)
    zeros8 = jnp.zeros((N, 8), jnp.float32)
    zerosD = jnp.zeros((N, D), jnp.float32)
    degp = _hist(s2d, ones8, zeros8)
    h_scaled = _matmul_scaled(nodes, W, b.reshape(1, D), degp)
    parts = _agg(h_scaled, s2d, r2d, zerosD)
    return _combine(parts)


# speedup vs baseline: 6.5883x; 6.5883x over previous
import jax
import jax.numpy as jnp
from jax import lax
from jax.experimental import pallas as pl
from jax.experimental.pallas import tpu as pltpu
from jax.experimental.pallas import tpu_sc as plsc

N = 10000
E = 320000
D = 128
BLK = 128
NB = E // BLK
NC = 2
NS = 16
NW = NC * NS
ROWS_PER_TILE = 624
TAIL0 = NS * ROWS_PER_TILE
TAIL = N - TAIL0
MAX_ITERS = (NB + NW - 1) // NW

_MESH = plsc.VectorSubcoreMesh(
    core_axis_name="c", subcore_axis_name="s", num_cores=NC, num_subcores=NS
)


def _slab_copy(src, dst, s):
    r0 = pl.multiple_of(s * ROWS_PER_TILE, 8)
    pltpu.sync_copy(src.at[pl.ds(r0, ROWS_PER_TILE)],
                    dst.at[pl.ds(r0, ROWS_PER_TILE)])

    @pl.when(s == NS - 1)
    def _():
        pltpu.sync_copy(src.at[pl.ds(TAIL0, TAIL)], dst.at[pl.ds(TAIL0, TAIL)])


def _hist_body(s2d_hbm, ones_hbm, zeros_hbm, out_hbm, sidx_v, ones_v, dacc):
    c = lax.axis_index("c")
    s = lax.axis_index("s")
    wid = s * NC + c
    _slab_copy(zeros_hbm, dacc, s)
    pltpu.sync_copy(ones_hbm, ones_v)
    plsc.subcore_barrier()

    def body(k, carry):
        j = wid + NW * k

        @pl.when(j < NB)
        def _():
            pltpu.sync_copy(s2d_hbm.at[j], sidx_v)
            pltpu.sync_copy(ones_v, dacc.at[sidx_v], add=True)

        return carry

    lax.fori_loop(0, MAX_ITERS, body, 0)
    plsc.subcore_barrier()
    _slab_copy(dacc, out_hbm.at[c], s)


_hist = pl.kernel(
    _hist_body,
    out_type=jax.ShapeDtypeStruct((NC, N, D), jnp.float32),
    mesh=_MESH,
    scratch_types=[
        pltpu.VMEM((BLK,), jnp.int32),
        pltpu.VMEM((BLK, D), jnp.float32),
        pltpu.VMEM_SHARED((N, D), jnp.float32),
    ],
)


def _agg_body(h_hbm, s2d_hbm, r2d_hbm, zeros_hbm, out_hbm,
              sidx_v, ridx_v, rows_v, sem, acc):
    c = lax.axis_index("c")
    s = lax.axis_index("s")
    wid = s * NC + c

    @pl.when(c == 0)
    def _():
        _slab_copy(h_hbm, acc, s)

    @pl.when(c == 1)
    def _():
        _slab_copy(zeros_hbm, acc, s)

    plsc.subcore_barrier()

    def body(k, carry):
        j = wid + NW * k

        @pl.when(j < NB)
        def _():
            pltpu.sync_copy(s2d_hbm.at[j], sidx_v)
            pltpu.sync_copy(r2d_hbm.at[j], ridx_v)
            pltpu.async_copy(h_hbm.at[sidx_v], rows_v, sem).wait()
            pltpu.sync_copy(rows_v, acc.at[ridx_v], add=True)

        return carry

    lax.fori_loop(0, MAX_ITERS, body, 0)
    plsc.subcore_barrier()
    _slab_copy(acc, out_hbm.at[c], s)


_agg = pl.kernel(
    _agg_body,
    out_type=jax.ShapeDtypeStruct((NC, N, D), jnp.float32),
    mesh=_MESH,
    scratch_types=[
        pltpu.VMEM((BLK,), jnp.int32),
        pltpu.VMEM((BLK,), jnp.int32),
        pltpu.VMEM((BLK, D), jnp.float32),
        pltpu.SemaphoreType.DMA,
        pltpu.VMEM_SHARED((N, D), jnp.float32),
    ],
)

_MM_ROWS = 2000


def _mm_body(nodes_ref, w_ref, b_ref, degp_ref, out_ref):
    inv = 1.0 / (degp_ref[0, :, 0:1] + degp_ref[1, :, 0:1] + 1.0)
    h = jnp.dot(nodes_ref[...], w_ref[...], preferred_element_type=jnp.float32)
    h = jnp.maximum(h + b_ref[...], 0.0)
    out_ref[...] = h * inv


def _matmul_scaled(nodes, w, b2d, degp):
    grid = N // _MM_ROWS
    return pl.pallas_call(
        _mm_body,
        grid=(grid,),
        in_specs=[
            pl.BlockSpec((_MM_ROWS, D), lambda i: (i, 0)),
            pl.BlockSpec((D, D), lambda i: (0, 0)),
            pl.BlockSpec((1, D), lambda i: (0, 0)),
            pl.BlockSpec((NC, _MM_ROWS, D), lambda i: (0, i, 0)),
        ],
        out_specs=pl.BlockSpec((_MM_ROWS, D), lambda i: (i, 0)),
        out_shape=jax.ShapeDtypeStruct((N, D), jnp.float32),
    )(nodes, w, b2d, degp)


def _add_body(p_ref, out_ref):
    out_ref[...] = p_ref[0] + p_ref[1]


def _combine(parts):
    grid = N // _MM_ROWS
    return pl.pallas_call(
        _add_body,
        grid=(grid,),
        in_specs=[pl.BlockSpec((NC, _MM_ROWS, D), lambda i: (0, i, 0))],
        out_specs=pl.BlockSpec((_MM_ROWS, D), lambda i: (i, 0)),
        out_shape=jax.ShapeDtypeStruct((N, D), jnp.float32),
    )(parts)


def kernel(nodes, senders, receivers, W, b):
    s2d = senders.reshape(NB, BLK)
    r2d = receivers.reshape(NB, BLK)
    onesD = jnp.ones((BLK, D), jnp.float32)
    zerosD = jnp.zeros((N, D), jnp.float32)
    degp = _hist(s2d, onesD, zerosD)
    h_scaled = _matmul_scaled(nodes, W, b.reshape(1, D), degp)
    parts = _agg(h_scaled, s2d, r2d, zerosD)
    return _combine(parts)


# speedup vs baseline: 8.0576x; 1.2230x over previous
import jax
import jax.numpy as jnp
from jax import lax
from jax.experimental import pallas as pl
from jax.experimental.pallas import tpu as pltpu
from jax.experimental.pallas import tpu_sc as plsc

N = 10000
E = 320000
D = 128
BLK = 128
NB = E // BLK
NC = 2
NS = 16
NW = NC * NS
ROWS_PER_TILE = 624
TAIL0 = NS * ROWS_PER_TILE
TAIL = N - TAIL0
MAX_ITERS = (NB + NW - 1) // NW

_MESH = plsc.VectorSubcoreMesh(
    core_axis_name="c", subcore_axis_name="s", num_cores=NC, num_subcores=NS
)


def _slab_copy(src, dst, s):
    r0 = pl.multiple_of(s * ROWS_PER_TILE, 8)
    pltpu.sync_copy(src.at[pl.ds(r0, ROWS_PER_TILE)],
                    dst.at[pl.ds(r0, ROWS_PER_TILE)])

    @pl.when(s == NS - 1)
    def _():
        pltpu.sync_copy(src.at[pl.ds(TAIL0, TAIL)], dst.at[pl.ds(TAIL0, TAIL)])


def _hist_body(s2d_hbm, ones_hbm, zeros_hbm, out_hbm, sidx_v, ones_v, dacc):
    c = lax.axis_index("c")
    s = lax.axis_index("s")
    wid = s * NC + c
    _slab_copy(zeros_hbm, dacc, s)
    pltpu.sync_copy(ones_hbm, ones_v)
    plsc.subcore_barrier()

    def body(k, carry):
        j = wid + NW * k

        @pl.when(j < NB)
        def _():
            pltpu.sync_copy(s2d_hbm.at[j], sidx_v)
            pltpu.sync_copy(ones_v, dacc.at[sidx_v], add=True)

        return carry

    lax.fori_loop(0, MAX_ITERS, body, 0)
    plsc.subcore_barrier()
    _slab_copy(dacc, out_hbm.at[c], s)


_hist = pl.kernel(
    _hist_body,
    out_type=jax.ShapeDtypeStruct((NC, N, D), jnp.float32),
    mesh=_MESH,
    scratch_types=[
        pltpu.VMEM((BLK,), jnp.int32),
        pltpu.VMEM((BLK, D), jnp.float32),
        pltpu.VMEM_SHARED((N, D), jnp.float32),
    ],
)


def _agg_body(h_hbm, s2d_hbm, r2d_hbm, zeros_hbm, out_hbm,
              sidx0, sidx1, ridx0, ridx1, rows0, rows1,
              ss0, ss1, sr0, sr1, sg0, sg1, acc):
    c = lax.axis_index("c")
    s = lax.axis_index("s")
    wid = s * NC + c
    sidx = (sidx0, sidx1)
    ridx = (ridx0, ridx1)
    rows = (rows0, rows1)
    sems_s = (ss0, ss1)
    sems_r = (sr0, sr1)
    sems_g = (sg0, sg1)

    @pl.when(c == 0)
    def _():
        _slab_copy(h_hbm, acc, s)

    @pl.when(c == 1)
    def _():
        _slab_copy(zeros_hbm, acc, s)

    plsc.subcore_barrier()

    pltpu.async_copy(s2d_hbm.at[wid], sidx[0], sems_s[0])
    pltpu.async_copy(r2d_hbm.at[wid], ridx[0], sems_r[0])

    def pair_body(k0, carry):
        for b in range(2):
            k = 2 * k0 + b
            jprev = wid + NW * (k - 1)
            jnext = wid + NW * (k + 1)
            j = wid + NW * k

            @pl.when((k >= 1) & (jprev < NB))
            def _():
                pltpu.make_async_copy(
                    h_hbm.at[sidx[1 - b]], rows[1 - b], sems_g[1 - b]).wait()
                pltpu.make_async_copy(
                    r2d_hbm.at[jprev], ridx[1 - b], sems_r[1 - b]).wait()
                pltpu.sync_copy(rows[1 - b], acc.at[ridx[1 - b]], add=True)

            @pl.when(jnext < NB)
            def _():
                pltpu.async_copy(s2d_hbm.at[jnext], sidx[1 - b], sems_s[1 - b])
                pltpu.async_copy(r2d_hbm.at[jnext], ridx[1 - b], sems_r[1 - b])

            @pl.when(j < NB)
            def _():
                pltpu.make_async_copy(
                    s2d_hbm.at[j], sidx[b], sems_s[b]).wait()
                pltpu.async_copy(h_hbm.at[sidx[b]], rows[b], sems_g[b])

        return carry

    lax.fori_loop(0, (MAX_ITERS + 2) // 2, pair_body, 0)
    plsc.subcore_barrier()
    _slab_copy(acc, out_hbm.at[c], s)


_agg = pl.kernel(
    _agg_body,
    out_type=jax.ShapeDtypeStruct((NC, N, D), jnp.float32),
    mesh=_MESH,
    scratch_types=[
        pltpu.VMEM((BLK,), jnp.int32),
        pltpu.VMEM((BLK,), jnp.int32),
        pltpu.VMEM((BLK,), jnp.int32),
        pltpu.VMEM((BLK,), jnp.int32),
        pltpu.VMEM((BLK, D), jnp.float32),
        pltpu.VMEM((BLK, D), jnp.float32),
        pltpu.SemaphoreType.DMA,
        pltpu.SemaphoreType.DMA,
        pltpu.SemaphoreType.DMA,
        pltpu.SemaphoreType.DMA,
        pltpu.SemaphoreType.DMA,
        pltpu.SemaphoreType.DMA,
        pltpu.VMEM_SHARED((N, D), jnp.float32),
    ],
)

_MM_ROWS = 2000


def _mm_body(nodes_ref, w_ref, b_ref, degp_ref, out_ref):
    inv = 1.0 / (degp_ref[0, :, 0:1] + degp_ref[1, :, 0:1] + 1.0)
    h = jnp.dot(nodes_ref[...], w_ref[...], preferred_element_type=jnp.float32)
    h = jnp.maximum(h + b_ref[...], 0.0)
    out_ref[...] = h * inv


def _matmul_scaled(nodes, w, b2d, degp):
    grid = N // _MM_ROWS
    return pl.pallas_call(
        _mm_body,
        grid=(grid,),
        in_specs=[
            pl.BlockSpec((_MM_ROWS, D), lambda i: (i, 0)),
            pl.BlockSpec((D, D), lambda i: (0, 0)),
            pl.BlockSpec((1, D), lambda i: (0, 0)),
            pl.BlockSpec((NC, _MM_ROWS, D), lambda i: (0, i, 0)),
        ],
        out_specs=pl.BlockSpec((_MM_ROWS, D), lambda i: (i, 0)),
        out_shape=jax.ShapeDtypeStruct((N, D), jnp.float32),
    )(nodes, w, b2d, degp)


def _add_body(p_ref, out_ref):
    out_ref[...] = p_ref[0] + p_ref[1]


def _combine(parts):
    grid = N // _MM_ROWS
    return pl.pallas_call(
        _add_body,
        grid=(grid,),
        in_specs=[pl.BlockSpec((NC, _MM_ROWS, D), lambda i: (0, i, 0))],
        out_specs=pl.BlockSpec((_MM_ROWS, D), lambda i: (i, 0)),
        out_shape=jax.ShapeDtypeStruct((N, D), jnp.float32),
    )(parts)


def kernel(nodes, senders, receivers, W, b):
    s2d = senders.reshape(NB, BLK)
    r2d = receivers.reshape(NB, BLK)
    onesD = jnp.ones((BLK, D), jnp.float32)
    zerosD = jnp.zeros((N, D), jnp.float32)
    degp = _hist(s2d, onesD, zerosD)
    h_scaled = _matmul_scaled(nodes, W, b.reshape(1, D), degp)
    parts = _agg(h_scaled, s2d, r2d, zerosD)
    return _combine(parts)


# speedup vs baseline: 10.1801x; 1.2634x over previous
import jax
import jax.numpy as jnp
from jax import lax
from jax.experimental import pallas as pl
from jax.experimental.pallas import tpu as pltpu
from jax.experimental.pallas import tpu_sc as plsc

N = 10000
E = 320000
D = 128
BLK = 128
NB = E // BLK
NC = 2
NS = 16
NW = NC * NS
ROWS_PER_TILE = 624
TAIL0 = NS * ROWS_PER_TILE
TAIL = N - TAIL0
MAX_ITERS = (NB + NW - 1) // NW

_MESH = plsc.VectorSubcoreMesh(
    core_axis_name="c", subcore_axis_name="s", num_cores=NC, num_subcores=NS
)


def _slab_copy(src, dst, s):
    r0 = pl.multiple_of(s * ROWS_PER_TILE, 8)
    pltpu.sync_copy(src.at[pl.ds(r0, ROWS_PER_TILE)],
                    dst.at[pl.ds(r0, ROWS_PER_TILE)])

    @pl.when(s == NS - 1)
    def _():
        pltpu.sync_copy(src.at[pl.ds(TAIL0, TAIL)], dst.at[pl.ds(TAIL0, TAIL)])


def _hist_body(s2d_hbm, out_hbm, sidx0, sidx1, ones_v, stage_t,
               si0, si1, dacc):
    c = lax.axis_index("c")
    s = lax.axis_index("s")
    wid = s * NC + c
    sidx = (sidx0, sidx1)
    sems = (si0, si1)
    r0 = pl.multiple_of(s * ROWS_PER_TILE, 8)

    for i in range(ROWS_PER_TILE // 16):
        stage_t[pl.ds(16 * i, 16)] = jnp.zeros((16,), jnp.float32)

    pltpu.sync_copy(stage_t, dacc.at[pl.ds(r0, ROWS_PER_TILE)])

    @pl.when(s == NS - 1)
    def _():
        pltpu.sync_copy(stage_t.at[pl.ds(0, TAIL)], dacc.at[pl.ds(TAIL0, TAIL)])

    for i in range(BLK // 16):
        ones_v[pl.ds(16 * i, 16)] = jnp.ones((16,), jnp.float32)

    plsc.subcore_barrier()

    pltpu.async_copy(s2d_hbm.at[wid], sidx[0], sems[0])

    def pair_body(k0, carry):
        for b in range(2):
            k = 2 * k0 + b
            jnext = wid + NW * (k + 1)
            j = wid + NW * k

            @pl.when(jnext < NB)
            def _():
                pltpu.async_copy(s2d_hbm.at[jnext], sidx[1 - b], sems[1 - b])

            @pl.when(j < NB)
            def _():
                pltpu.make_async_copy(
                    s2d_hbm.at[j], sidx[b], sems[b]).wait()
                pltpu.sync_copy(ones_v, dacc.at[sidx[b]], add=True)

        return carry

    lax.fori_loop(0, (MAX_ITERS + 1) // 2, pair_body, 0)
    plsc.subcore_barrier()
    base = pl.multiple_of(c * N + r0, 8)
    pltpu.sync_copy(dacc.at[pl.ds(r0, ROWS_PER_TILE)], stage_t)
    pltpu.sync_copy(stage_t, out_hbm.at[pl.ds(base, ROWS_PER_TILE)])

    @pl.when(s == NS - 1)
    def _():
        pltpu.sync_copy(dacc.at[pl.ds(TAIL0, TAIL)], stage_t.at[pl.ds(0, TAIL)])
        pltpu.sync_copy(stage_t.at[pl.ds(0, TAIL)],
                        out_hbm.at[pl.ds(pl.multiple_of(c * N + TAIL0, 8), TAIL)])


_hist = pl.kernel(
    _hist_body,
    out_type=jax.ShapeDtypeStruct((NC * N,), jnp.float32),
    mesh=_MESH,
    scratch_types=[
        pltpu.VMEM((BLK,), jnp.int32),
        pltpu.VMEM((BLK,), jnp.int32),
        pltpu.VMEM((BLK,), jnp.float32),
        pltpu.VMEM((ROWS_PER_TILE,), jnp.float32),
        pltpu.SemaphoreType.DMA,
        pltpu.SemaphoreType.DMA,
        pltpu.VMEM_SHARED((N,), jnp.float32),
    ],
)


def _agg_body(h_hbm, s2d_hbm, r2d_hbm, zeros_hbm, out_hbm,
              sidx0, sidx1, ridx0, ridx1, rows0, rows1,
              ss0, ss1, sr0, sr1, sg0, sg1, acc):
    c = lax.axis_index("c")
    s = lax.axis_index("s")
    wid = s * NC + c
    sidx = (sidx0, sidx1)
    ridx = (ridx0, ridx1)
    rows = (rows0, rows1)
    sems_s = (ss0, ss1)
    sems_r = (sr0, sr1)
    sems_g = (sg0, sg1)

    @pl.when(c == 0)
    def _():
        _slab_copy(h_hbm, acc, s)

    @pl.when(c == 1)
    def _():
        _slab_copy(zeros_hbm, acc, s)

    plsc.subcore_barrier()

    pltpu.async_copy(s2d_hbm.at[wid], sidx[0], sems_s[0])
    pltpu.async_copy(r2d_hbm.at[wid], ridx[0], sems_r[0])

    def pair_body(k0, carry):
        for b in range(2):
            k = 2 * k0 + b
            jprev = wid + NW * (k - 1)
            jnext = wid + NW * (k + 1)
            j = wid + NW * k

            @pl.when((k >= 1) & (jprev < NB))
            def _():
                pltpu.make_async_copy(
                    h_hbm.at[sidx[1 - b]], rows[1 - b], sems_g[1 - b]).wait()
                pltpu.make_async_copy(
                    r2d_hbm.at[jprev], ridx[1 - b], sems_r[1 - b]).wait()
                pltpu.sync_copy(rows[1 - b], acc.at[ridx[1 - b]], add=True)

            @pl.when(jnext < NB)
            def _():
                pltpu.async_copy(s2d_hbm.at[jnext], sidx[1 - b], sems_s[1 - b])
                pltpu.async_copy(r2d_hbm.at[jnext], ridx[1 - b], sems_r[1 - b])

            @pl.when(j < NB)
            def _():
                pltpu.make_async_copy(
                    s2d_hbm.at[j], sidx[b], sems_s[b]).wait()
                pltpu.async_copy(h_hbm.at[sidx[b]], rows[b], sems_g[b])

        return carry

    lax.fori_loop(0, (MAX_ITERS + 2) // 2, pair_body, 0)
    plsc.subcore_barrier()
    _slab_copy(acc, out_hbm.at[c], s)


_agg = pl.kernel(
    _agg_body,
    out_type=jax.ShapeDtypeStruct((NC, N, D), jnp.float32),
    mesh=_MESH,
    scratch_types=[
        pltpu.VMEM((BLK,), jnp.int32),
        pltpu.VMEM((BLK,), jnp.int32),
        pltpu.VMEM((BLK,), jnp.int32),
        pltpu.VMEM((BLK,), jnp.int32),
        pltpu.VMEM((BLK, D), jnp.float32),
        pltpu.VMEM((BLK, D), jnp.float32),
        pltpu.SemaphoreType.DMA,
        pltpu.SemaphoreType.DMA,
        pltpu.SemaphoreType.DMA,
        pltpu.SemaphoreType.DMA,
        pltpu.SemaphoreType.DMA,
        pltpu.SemaphoreType.DMA,
        pltpu.VMEM_SHARED((N, D), jnp.float32),
    ],
)

_MM_ROWS = 2000


def _mm_body(nodes_ref, w_ref, b_ref, degp_ref, out_ref):
    inv = 1.0 / (degp_ref[0] + degp_ref[1] + 1.0)
    h = jnp.dot(nodes_ref[...], w_ref[...], preferred_element_type=jnp.float32)
    h = jnp.maximum(h + b_ref[...], 0.0)
    out_ref[...] = h * inv


def _matmul_scaled(nodes, w, b2d, degp):
    grid = N // _MM_ROWS
    return pl.pallas_call(
        _mm_body,
        grid=(grid,),
        in_specs=[
            pl.BlockSpec((_MM_ROWS, D), lambda i: (i, 0)),
            pl.BlockSpec((D, D), lambda i: (0, 0)),
            pl.BlockSpec((1, D), lambda i: (0, 0)),
            pl.BlockSpec((NC, _MM_ROWS, 1), lambda i: (0, i, 0)),
        ],
        out_specs=pl.BlockSpec((_MM_ROWS, D), lambda i: (i, 0)),
        out_shape=jax.ShapeDtypeStruct((N, D), jnp.float32),
    )(nodes, w, b2d, degp)


def _add_body(p_ref, out_ref):
    out_ref[...] = p_ref[0] + p_ref[1]


def _combine(parts):
    grid = N // _MM_ROWS
    return pl.pallas_call(
        _add_body,
        grid=(grid,),
        in_specs=[pl.BlockSpec((NC, _MM_ROWS, D), lambda i: (0, i, 0))],
        out_specs=pl.BlockSpec((_MM_ROWS, D), lambda i: (i, 0)),
        out_shape=jax.ShapeDtypeStruct((N, D), jnp.float32),
    )(parts)


def kernel(nodes, senders, receivers, W, b):
    s2d = senders.reshape(NB, BLK)
    r2d = receivers.reshape(NB, BLK)
    zerosD = jnp.zeros((N, D), jnp.float32)
    degflat = _hist(s2d)
    degp3 = degflat.reshape(NC, N, 1)
    h_scaled = _matmul_scaled(nodes, W, b.reshape(1, D), degp3)
    parts = _agg(h_scaled, s2d, r2d, zerosD)
    return _combine(parts)


# speedup vs baseline: 12.4735x; 1.2253x over previous
import jax
import jax.numpy as jnp
from jax import lax
from jax.experimental import pallas as pl
from jax.experimental.pallas import tpu as pltpu
from jax.experimental.pallas import tpu_sc as plsc

N = 10000
E = 320000
D = 128
BLK = 128
NB = E // BLK
NC = 2
NS = 16
NW = NC * NS
ROWS_PER_TILE = 624
TAIL0 = NS * ROWS_PER_TILE
TAIL = N - TAIL0
MAX_ITERS = (NB + NW - 1) // NW

_MESH = plsc.VectorSubcoreMesh(
    core_axis_name="c", subcore_axis_name="s", num_cores=NC, num_subcores=NS
)


def _slab_copy(src, dst, s):
    r0 = pl.multiple_of(s * ROWS_PER_TILE, 8)
    pltpu.sync_copy(src.at[pl.ds(r0, ROWS_PER_TILE)],
                    dst.at[pl.ds(r0, ROWS_PER_TILE)])

    @pl.when(s == NS - 1)
    def _():
        pltpu.sync_copy(src.at[pl.ds(TAIL0, TAIL)], dst.at[pl.ds(TAIL0, TAIL)])


def _hist_body(s2d_hbm, out_hbm, sidx0, sidx1, ones_v, stage_t,
               si0, si1, dacc):
    c = lax.axis_index("c")
    s = lax.axis_index("s")
    wid = s * NC + c
    sidx = (sidx0, sidx1)
    sems = (si0, si1)
    r0 = pl.multiple_of(s * ROWS_PER_TILE, 8)

    for i in range(ROWS_PER_TILE // 16):
        stage_t[pl.ds(16 * i, 16)] = jnp.zeros((16,), jnp.float32)

    pltpu.sync_copy(stage_t, dacc.at[pl.ds(r0, ROWS_PER_TILE)])

    @pl.when(s == NS - 1)
    def _():
        pltpu.sync_copy(stage_t.at[pl.ds(0, TAIL)], dacc.at[pl.ds(TAIL0, TAIL)])

    for i in range(BLK // 16):
        ones_v[pl.ds(16 * i, 16)] = jnp.ones((16,), jnp.float32)

    plsc.subcore_barrier()

    pltpu.async_copy(s2d_hbm.at[wid], sidx[0], sems[0])

    def pair_body(k0, carry):
        for b in range(2):
            k = 2 * k0 + b
            jnext = wid + NW * (k + 1)
            j = wid + NW * k

            @pl.when(jnext < NB)
            def _():
                pltpu.async_copy(s2d_hbm.at[jnext], sidx[1 - b], sems[1 - b])

            @pl.when(j < NB)
            def _():
                pltpu.make_async_copy(
                    s2d_hbm.at[j], sidx[b], sems[b]).wait()
                pltpu.sync_copy(ones_v, dacc.at[sidx[b]], add=True)

        return carry

    lax.fori_loop(0, (MAX_ITERS + 1) // 2, pair_body, 0)
    plsc.subcore_barrier()
    base = pl.multiple_of(c * N + r0, 8)
    pltpu.sync_copy(dacc.at[pl.ds(r0, ROWS_PER_TILE)], stage_t)
    pltpu.sync_copy(stage_t, out_hbm.at[pl.ds(base, ROWS_PER_TILE)])

    @pl.when(s == NS - 1)
    def _():
        pltpu.sync_copy(dacc.at[pl.ds(TAIL0, TAIL)], stage_t.at[pl.ds(0, TAIL)])
        pltpu.sync_copy(stage_t.at[pl.ds(0, TAIL)],
                        out_hbm.at[pl.ds(pl.multiple_of(c * N + TAIL0, 8), TAIL)])


_hist = pl.kernel(
    _hist_body,
    out_type=jax.ShapeDtypeStruct((NC * N,), jnp.float32),
    mesh=_MESH,
    scratch_types=[
        pltpu.VMEM((BLK,), jnp.int32),
        pltpu.VMEM((BLK,), jnp.int32),
        pltpu.VMEM((BLK,), jnp.float32),
        pltpu.VMEM((ROWS_PER_TILE,), jnp.float32),
        pltpu.SemaphoreType.DMA,
        pltpu.SemaphoreType.DMA,
        pltpu.VMEM_SHARED((N,), jnp.float32),
    ],
)


_DEPTH = 4
_RDEPTH = 2


def _agg_body(h_hbm, s2d_hbm, r2d_hbm, zeros_hbm, out_hbm, *refs):
    sidx = refs[0:_DEPTH]
    ridx = refs[_DEPTH:2 * _DEPTH]
    rows = refs[2 * _DEPTH:2 * _DEPTH + _RDEPTH]
    sems_s = refs[2 * _DEPTH + _RDEPTH:3 * _DEPTH + _RDEPTH]
    sems_r = refs[3 * _DEPTH + _RDEPTH:4 * _DEPTH + _RDEPTH]
    sems_g = refs[4 * _DEPTH + _RDEPTH:4 * _DEPTH + 2 * _RDEPTH]
    sems_c = refs[4 * _DEPTH + 2 * _RDEPTH:4 * _DEPTH + 3 * _RDEPTH]
    acc = refs[4 * _DEPTH + 3 * _RDEPTH]
    c = lax.axis_index("c")
    s = lax.axis_index("s")
    wid = s * NC + c

    @pl.when(c == 0)
    def _():
        _slab_copy(h_hbm, acc, s)

    @pl.when(c == 1)
    def _():
        _slab_copy(zeros_hbm, acc, s)

    plsc.subcore_barrier()

    pltpu.async_copy(s2d_hbm.at[wid], sidx[0], sems_s[0])
    pltpu.async_copy(r2d_hbm.at[wid], ridx[0], sems_r[0])

    @pl.when(wid + NW < NB)
    def _():
        pltpu.async_copy(s2d_hbm.at[wid + NW], sidx[1], sems_s[1])
        pltpu.async_copy(r2d_hbm.at[wid + NW], ridx[1], sems_r[1])

    def quad_body(k0, carry):
        for b in range(_DEPTH):
            k = _DEPTH * k0 + b
            bw = (b - 2) % _DEPTH
            ba = (b + 2) % _DEPTH
            bg = (b - 1) % _DEPTH
            rw = (b - 2) % _RDEPTH
            rg = (b - 1) % _RDEPTH

            @pl.when((k >= 2) & (wid + NW * (k - 2) < NB))
            def _():
                pltpu.make_async_copy(
                    rows[rw], acc.at[ridx[bw]], sems_c[rw]).wait()

            @pl.when(wid + NW * (k + 2) < NB)
            def _():
                pltpu.async_copy(s2d_hbm.at[wid + NW * (k + 2)],
                                 sidx[ba], sems_s[ba])
                pltpu.async_copy(r2d_hbm.at[wid + NW * (k + 2)],
                                 ridx[ba], sems_r[ba])

            @pl.when((k >= 1) & (wid + NW * (k - 1) < NB))
            def _():
                pltpu.make_async_copy(
                    h_hbm.at[sidx[bg]], rows[rg], sems_g[rg]).wait()
                pltpu.make_async_copy(
                    r2d_hbm.at[wid + NW * (k - 1)], ridx[bg],
                    sems_r[bg]).wait()
                pltpu.async_copy(rows[rg], acc.at[ridx[bg]], sems_c[rg],
                                 add=True)

            @pl.when(wid + NW * k < NB)
            def _():
                pltpu.make_async_copy(
                    s2d_hbm.at[wid + NW * k], sidx[b], sems_s[b]).wait()
                pltpu.async_copy(h_hbm.at[sidx[b]],
                                 rows[b % _RDEPTH], sems_g[b % _RDEPTH])

        return carry

    lax.fori_loop(0, (MAX_ITERS + 2 + _DEPTH - 1) // _DEPTH, quad_body, 0)
    plsc.subcore_barrier()
    _slab_copy(acc, out_hbm.at[c], s)


_agg = pl.kernel(
    _agg_body,
    out_type=jax.ShapeDtypeStruct((NC, N, D), jnp.float32),
    mesh=_MESH,
    scratch_types=(
        [pltpu.VMEM((BLK,), jnp.int32)] * (2 * _DEPTH)
        + [pltpu.VMEM((BLK, D), jnp.float32)] * _RDEPTH
        + [pltpu.SemaphoreType.DMA] * (2 * _DEPTH + 2 * _RDEPTH)
        + [pltpu.VMEM_SHARED((N, D), jnp.float32)]
    ),
)

_MM_ROWS = 2000


def _mm_body(nodes_ref, w_ref, b_ref, degp_ref, out_ref):
    inv = 1.0 / (degp_ref[0] + degp_ref[1] + 1.0)
    h = jnp.dot(nodes_ref[...], w_ref[...], preferred_element_type=jnp.float32)
    h = jnp.maximum(h + b_ref[...], 0.0)
    out_ref[...] = h * inv


def _matmul_scaled(nodes, w, b2d, degp):
    grid = N // _MM_ROWS
    return pl.pallas_call(
        _mm_body,
        grid=(grid,),
        in_specs=[
            pl.BlockSpec((_MM_ROWS, D), lambda i: (i, 0)),
            pl.BlockSpec((D, D), lambda i: (0, 0)),
            pl.BlockSpec((1, D), lambda i: (0, 0)),
            pl.BlockSpec((NC, _MM_ROWS, 1), lambda i: (0, i, 0)),
        ],
        out_specs=pl.BlockSpec((_MM_ROWS, D), lambda i: (i, 0)),
        out_shape=jax.ShapeDtypeStruct((N, D), jnp.float32),
    )(nodes, w, b2d, degp)


def _add_body(p_ref, out_ref):
    out_ref[...] = p_ref[0] + p_ref[1]


def _combine(parts):
    grid = N // _MM_ROWS
    return pl.pallas_call(
        _add_body,
        grid=(grid,),
        in_specs=[pl.BlockSpec((NC, _MM_ROWS, D), lambda i: (0, i, 0))],
        out_specs=pl.BlockSpec((_MM_ROWS, D), lambda i: (i, 0)),
        out_shape=jax.ShapeDtypeStruct((N, D), jnp.float32),
    )(parts)


def kernel(nodes, senders, receivers, W, b):
    s2d = senders.reshape(NB, BLK)
    r2d = receivers.reshape(NB, BLK)
    zerosD = jnp.zeros((N, D), jnp.float32)
    degflat = _hist(s2d)
    degp3 = degflat.reshape(NC, N, 1)
    h_scaled = _matmul_scaled(nodes, W, b.reshape(1, D), degp3)
    parts = _agg(h_scaled, s2d, r2d, zerosD)
    return _combine(parts)


# speedup vs baseline: 14.0883x; 1.1295x over previous
import jax
import jax.numpy as jnp
from jax import lax
from jax.experimental import pallas as pl
from jax.experimental.pallas import tpu as pltpu
from jax.experimental.pallas import tpu_sc as plsc

N = 10000
E = 320000
D = 128
BLK = 128
NB = E // BLK
NC = 2
NS = 16
NW = NC * NS
ROWS_PER_TILE = 624
TAIL0 = NS * ROWS_PER_TILE
TAIL = N - TAIL0
MAX_ITERS = (NB + NW - 1) // NW

_MESH = plsc.VectorSubcoreMesh(
    core_axis_name="c", subcore_axis_name="s", num_cores=NC, num_subcores=NS
)


def _slab_copy(src, dst, s):
    r0 = pl.multiple_of(s * ROWS_PER_TILE, 8)
    pltpu.sync_copy(src.at[pl.ds(r0, ROWS_PER_TILE)],
                    dst.at[pl.ds(r0, ROWS_PER_TILE)])

    @pl.when(s == NS - 1)
    def _():
        pltpu.sync_copy(src.at[pl.ds(TAIL0, TAIL)], dst.at[pl.ds(TAIL0, TAIL)])


def _hist_body(s2d_hbm, out_hbm, sidx0, sidx1, ones_v, stage_t,
               si0, si1, dacc):
    c = lax.axis_index("c")
    s = lax.axis_index("s")
    wid = s * NC + c
    sidx = (sidx0, sidx1)
    sems = (si0, si1)
    r0 = pl.multiple_of(s * ROWS_PER_TILE, 8)

    for i in range(ROWS_PER_TILE // 16):
        stage_t[pl.ds(16 * i, 16)] = jnp.zeros((16,), jnp.float32)

    pltpu.sync_copy(stage_t, dacc.at[pl.ds(r0, ROWS_PER_TILE)])

    @pl.when(s == NS - 1)
    def _():
        pltpu.sync_copy(stage_t.at[pl.ds(0, TAIL)], dacc.at[pl.ds(TAIL0, TAIL)])

    for i in range(BLK // 16):
        ones_v[pl.ds(16 * i, 16)] = jnp.ones((16,), jnp.float32)

    plsc.subcore_barrier()

    pltpu.async_copy(s2d_hbm.at[wid], sidx[0], sems[0])

    def pair_body(k0, carry):
        for b in range(2):
            k = 2 * k0 + b
            jnext = wid + NW * (k + 1)
            j = wid + NW * k

            @pl.when(jnext < NB)
            def _():
                pltpu.async_copy(s2d_hbm.at[jnext], sidx[1 - b], sems[1 - b])

            @pl.when(j < NB)
            def _():
                pltpu.make_async_copy(
                    s2d_hbm.at[j], sidx[b], sems[b]).wait()
                pltpu.sync_copy(ones_v, dacc.at[sidx[b]], add=True)

        return carry

    lax.fori_loop(0, (MAX_ITERS + 1) // 2, pair_body, 0)
    plsc.subcore_barrier()
    base = pl.multiple_of(c * N + r0, 8)
    pltpu.sync_copy(dacc.at[pl.ds(r0, ROWS_PER_TILE)], stage_t)
    pltpu.sync_copy(stage_t, out_hbm.at[pl.ds(base, ROWS_PER_TILE)])

    @pl.when(s == NS - 1)
    def _():
        pltpu.sync_copy(dacc.at[pl.ds(TAIL0, TAIL)], stage_t.at[pl.ds(0, TAIL)])
        pltpu.sync_copy(stage_t.at[pl.ds(0, TAIL)],
                        out_hbm.at[pl.ds(pl.multiple_of(c * N + TAIL0, 8), TAIL)])


_hist = pl.kernel(
    _hist_body,
    out_type=jax.ShapeDtypeStruct((NC * N,), jnp.float32),
    mesh=_MESH,
    scratch_types=[
        pltpu.VMEM((BLK,), jnp.int32),
        pltpu.VMEM((BLK,), jnp.int32),
        pltpu.VMEM((BLK,), jnp.float32),
        pltpu.VMEM((ROWS_PER_TILE,), jnp.float32),
        pltpu.SemaphoreType.DMA,
        pltpu.SemaphoreType.DMA,
        pltpu.VMEM_SHARED((N,), jnp.float32),
    ],
)


_DEPTH = 4
_RDEPTH = 2


def _agg_body(h_hbm, s2d_hbm, r2d_hbm, zeros_hbm, out_hbm, *refs):
    sidx = refs[0:_DEPTH]
    ridx = refs[_DEPTH:2 * _DEPTH]
    rows = refs[2 * _DEPTH:2 * _DEPTH + _RDEPTH]
    sems_s = refs[2 * _DEPTH + _RDEPTH:3 * _DEPTH + _RDEPTH]
    sems_r = refs[3 * _DEPTH + _RDEPTH:4 * _DEPTH + _RDEPTH]
    sems_g = refs[4 * _DEPTH + _RDEPTH:4 * _DEPTH + 2 * _RDEPTH]
    sems_c = refs[4 * _DEPTH + 2 * _RDEPTH:4 * _DEPTH + 3 * _RDEPTH]
    acc = refs[4 * _DEPTH + 3 * _RDEPTH]
    c = lax.axis_index("c")
    s = lax.axis_index("s")
    wid = s * NC + c

    @pl.when(c == 0)
    def _():
        _slab_copy(h_hbm, acc, s)

    @pl.when(c == 1)
    def _():
        _slab_copy(zeros_hbm, acc, s)

    plsc.subcore_barrier()

    pltpu.async_copy(s2d_hbm.at[wid], sidx[0], sems_s[0])
    pltpu.async_copy(r2d_hbm.at[wid], ridx[0], sems_r[0])

    @pl.when(wid + NW < NB)
    def _():
        pltpu.async_copy(s2d_hbm.at[wid + NW], sidx[1], sems_s[1])
        pltpu.async_copy(r2d_hbm.at[wid + NW], ridx[1], sems_r[1])

    def quad_body(k0, carry):
        for b in range(_DEPTH):
            k = _DEPTH * k0 + b
            bw = (b - 2) % _DEPTH
            ba = (b + 2) % _DEPTH
            bg = (b - 1) % _DEPTH
            rw = (b - 2) % _RDEPTH
            rg = (b - 1) % _RDEPTH

            @pl.when((k >= 2) & (wid + NW * (k - 2) < NB))
            def _():
                pltpu.make_async_copy(
                    rows[rw], acc.at[ridx[bw]], sems_c[rw]).wait()

            @pl.when(wid + NW * (k + 2) < NB)
            def _():
                pltpu.async_copy(s2d_hbm.at[wid + NW * (k + 2)],
                                 sidx[ba], sems_s[ba])
                pltpu.async_copy(r2d_hbm.at[wid + NW * (k + 2)],
                                 ridx[ba], sems_r[ba])

            @pl.when(wid + NW * k < NB)
            def _():
                pltpu.make_async_copy(
                    s2d_hbm.at[wid + NW * k], sidx[b], sems_s[b]).wait()
                pltpu.async_copy(h_hbm.at[sidx[b]],
                                 rows[b % _RDEPTH], sems_g[b % _RDEPTH])

            @pl.when((k >= 1) & (wid + NW * (k - 1) < NB))
            def _():
                pltpu.make_async_copy(
                    h_hbm.at[sidx[bg]], rows[rg], sems_g[rg]).wait()
                pltpu.make_async_copy(
                    r2d_hbm.at[wid + NW * (k - 1)], ridx[bg],
                    sems_r[bg]).wait()
                pltpu.async_copy(rows[rg], acc.at[ridx[bg]], sems_c[rg],
                                 add=True)

        return carry

    lax.fori_loop(0, (MAX_ITERS + 2 + _DEPTH - 1) // _DEPTH, quad_body, 0)
    plsc.subcore_barrier()
    _slab_copy(acc, out_hbm.at[c], s)


_agg = pl.kernel(
    _agg_body,
    out_type=jax.ShapeDtypeStruct((NC, N, D), jnp.float32),
    mesh=_MESH,
    scratch_types=(
        [pltpu.VMEM((BLK,), jnp.int32)] * (2 * _DEPTH)
        + [pltpu.VMEM((BLK, D), jnp.float32)] * _RDEPTH
        + [pltpu.SemaphoreType.DMA] * (2 * _DEPTH + 2 * _RDEPTH)
        + [pltpu.VMEM_SHARED((N, D), jnp.float32)]
    ),
)

_MM_ROWS = 2000


def _mm_body(nodes_ref, w_ref, b_ref, degp_ref, out_ref):
    inv = 1.0 / (degp_ref[0] + degp_ref[1] + 1.0)
    h = jnp.dot(nodes_ref[...], w_ref[...], preferred_element_type=jnp.float32)
    h = jnp.maximum(h + b_ref[...], 0.0)
    out_ref[...] = h * inv


def _matmul_scaled(nodes, w, b2d, degp):
    grid = N // _MM_ROWS
    return pl.pallas_call(
        _mm_body,
        grid=(grid,),
        in_specs=[
            pl.BlockSpec((_MM_ROWS, D), lambda i: (i, 0)),
            pl.BlockSpec((D, D), lambda i: (0, 0)),
            pl.BlockSpec((1, D), lambda i: (0, 0)),
            pl.BlockSpec((NC, _MM_ROWS, 1), lambda i: (0, i, 0)),
        ],
        out_specs=pl.BlockSpec((_MM_ROWS, D), lambda i: (i, 0)),
        out_shape=jax.ShapeDtypeStruct((N, D), jnp.float32),
    )(nodes, w, b2d, degp)


def _add_body(p_ref, out_ref):
    out_ref[...] = p_ref[0] + p_ref[1]


def _combine(parts):
    grid = N // _MM_ROWS
    return pl.pallas_call(
        _add_body,
        grid=(grid,),
        in_specs=[pl.BlockSpec((NC, _MM_ROWS, D), lambda i: (0, i, 0))],
        out_specs=pl.BlockSpec((_MM_ROWS, D), lambda i: (i, 0)),
        out_shape=jax.ShapeDtypeStruct((N, D), jnp.float32),
    )(parts)


def kernel(nodes, senders, receivers, W, b):
    s2d = senders.reshape(NB, BLK)
    r2d = receivers.reshape(NB, BLK)
    zerosD = jnp.zeros((N, D), jnp.float32)
    degflat = _hist(s2d)
    degp3 = degflat.reshape(NC, N, 1)
    h_scaled = _matmul_scaled(nodes, W, b.reshape(1, D), degp3)
    parts = _agg(h_scaled, s2d, r2d, zerosD)
    return _combine(parts)


# speedup vs baseline: 15.1466x; 1.0751x over previous
import jax
import jax.numpy as jnp
from jax import lax
from jax.experimental import pallas as pl
from jax.experimental.pallas import tpu as pltpu
from jax.experimental.pallas import tpu_sc as plsc

N = 10000
E = 320000
D = 128
BLK = 128
NB = E // BLK
NC = 2
NS = 16
NW = NC * NS
ROWS_PER_TILE = 624
TAIL0 = NS * ROWS_PER_TILE
TAIL = N - TAIL0
MAX_ITERS = (NB + NW - 1) // NW

_MESH = plsc.VectorSubcoreMesh(
    core_axis_name="c", subcore_axis_name="s", num_cores=NC, num_subcores=NS
)


def _slab_copy(src, dst, s):
    r0 = pl.multiple_of(s * ROWS_PER_TILE, 8)
    pltpu.sync_copy(src.at[pl.ds(r0, ROWS_PER_TILE)],
                    dst.at[pl.ds(r0, ROWS_PER_TILE)])

    @pl.when(s == NS - 1)
    def _():
        pltpu.sync_copy(src.at[pl.ds(TAIL0, TAIL)], dst.at[pl.ds(TAIL0, TAIL)])


def _hist_body(s2d_hbm, out_hbm, sidx0, sidx1, ones_v, stage_t,
               si0, si1, dacc):
    c = lax.axis_index("c")
    s = lax.axis_index("s")
    wid = s * NC + c
    sidx = (sidx0, sidx1)
    sems = (si0, si1)
    r0 = pl.multiple_of(s * ROWS_PER_TILE, 8)

    for i in range(ROWS_PER_TILE // 16):
        stage_t[pl.ds(16 * i, 16)] = jnp.zeros((16,), jnp.float32)

    pltpu.sync_copy(stage_t, dacc.at[pl.ds(r0, ROWS_PER_TILE)])

    @pl.when(s == NS - 1)
    def _():
        pltpu.sync_copy(stage_t.at[pl.ds(0, TAIL)], dacc.at[pl.ds(TAIL0, TAIL)])

    for i in range(BLK // 16):
        ones_v[pl.ds(16 * i, 16)] = jnp.ones((16,), jnp.float32)

    plsc.subcore_barrier()

    pltpu.async_copy(s2d_hbm.at[wid], sidx[0], sems[0])

    def pair_body(k0, carry):
        for b in range(2):
            k = 2 * k0 + b
            jnext = wid + NW * (k + 1)
            j = wid + NW * k

            @pl.when(jnext < NB)
            def _():
                pltpu.async_copy(s2d_hbm.at[jnext], sidx[1 - b], sems[1 - b])

            @pl.when(j < NB)
            def _():
                pltpu.make_async_copy(
                    s2d_hbm.at[j], sidx[b], sems[b]).wait()
                pltpu.sync_copy(ones_v, dacc.at[sidx[b]], add=True)

        return carry

    lax.fori_loop(0, (MAX_ITERS + 1) // 2, pair_body, 0)
    plsc.subcore_barrier()
    base = pl.multiple_of(c * N + r0, 8)
    pltpu.sync_copy(dacc.at[pl.ds(r0, ROWS_PER_TILE)], stage_t)
    pltpu.sync_copy(stage_t, out_hbm.at[pl.ds(base, ROWS_PER_TILE)])

    @pl.when(s == NS - 1)
    def _():
        pltpu.sync_copy(dacc.at[pl.ds(TAIL0, TAIL)], stage_t.at[pl.ds(0, TAIL)])
        pltpu.sync_copy(stage_t.at[pl.ds(0, TAIL)],
                        out_hbm.at[pl.ds(pl.multiple_of(c * N + TAIL0, 8), TAIL)])


_hist = pl.kernel(
    _hist_body,
    out_type=jax.ShapeDtypeStruct((NC * N,), jnp.float32),
    mesh=_MESH,
    scratch_types=[
        pltpu.VMEM((BLK,), jnp.int32),
        pltpu.VMEM((BLK,), jnp.int32),
        pltpu.VMEM((BLK,), jnp.float32),
        pltpu.VMEM((ROWS_PER_TILE,), jnp.float32),
        pltpu.SemaphoreType.DMA,
        pltpu.SemaphoreType.DMA,
        pltpu.VMEM_SHARED((N,), jnp.float32),
    ],
)


_DEPTH = 6
_RDEPTH = 3


def _agg_body(h_hbm, s2d_hbm, r2d_hbm, zeros_hbm, out_hbm, *refs):
    sidx = refs[0:_DEPTH]
    ridx = refs[_DEPTH:2 * _DEPTH]
    rows = refs[2 * _DEPTH:2 * _DEPTH + _RDEPTH]
    sems_s = refs[2 * _DEPTH + _RDEPTH:3 * _DEPTH + _RDEPTH]
    sems_r = refs[3 * _DEPTH + _RDEPTH:4 * _DEPTH + _RDEPTH]
    sems_g = refs[4 * _DEPTH + _RDEPTH:4 * _DEPTH + 2 * _RDEPTH]
    sems_c = refs[4 * _DEPTH + 2 * _RDEPTH:4 * _DEPTH + 3 * _RDEPTH]
    acc = refs[4 * _DEPTH + 3 * _RDEPTH]
    c = lax.axis_index("c")
    s = lax.axis_index("s")
    wid = s * NC + c

    @pl.when(c == 0)
    def _():
        _slab_copy(h_hbm, acc, s)

    @pl.when(c == 1)
    def _():
        _slab_copy(zeros_hbm, acc, s)

    plsc.subcore_barrier()

    for p in range(3):
        @pl.when(wid + NW * p < NB)
        def _():
            pltpu.async_copy(s2d_hbm.at[wid + NW * p], sidx[p], sems_s[p])
            pltpu.async_copy(r2d_hbm.at[wid + NW * p], ridx[p], sems_r[p])

    def ring_body(k0, carry):
        for b in range(_DEPTH):
            k = _DEPTH * k0 + b
            bw = (b - 3) % _DEPTH
            ba = (b + 3) % _DEPTH
            bg = (b - 2) % _DEPTH
            rw = (b - 3) % _RDEPTH
            rg = (b - 2) % _RDEPTH

            @pl.when((k >= 3) & (wid + NW * (k - 3) < NB))
            def _():
                pltpu.make_async_copy(
                    rows[rw], acc.at[ridx[bw]], sems_c[rw]).wait()

            @pl.when(wid + NW * (k + 3) < NB)
            def _():
                pltpu.async_copy(s2d_hbm.at[wid + NW * (k + 3)],
                                 sidx[ba], sems_s[ba])
                pltpu.async_copy(r2d_hbm.at[wid + NW * (k + 3)],
                                 ridx[ba], sems_r[ba])

            @pl.when(wid + NW * k < NB)
            def _():
                pltpu.make_async_copy(
                    s2d_hbm.at[wid + NW * k], sidx[b], sems_s[b]).wait()
                pltpu.async_copy(h_hbm.at[sidx[b]],
                                 rows[b % _RDEPTH], sems_g[b % _RDEPTH])

            @pl.when((k >= 2) & (wid + NW * (k - 2) < NB))
            def _():
                pltpu.make_async_copy(
                    h_hbm.at[sidx[bg]], rows[rg], sems_g[rg]).wait()
                pltpu.make_async_copy(
                    r2d_hbm.at[wid + NW * (k - 2)], ridx[bg],
                    sems_r[bg]).wait()
                pltpu.async_copy(rows[rg], acc.at[ridx[bg]], sems_c[rg],
                                 add=True)

        return carry

    lax.fori_loop(0, (MAX_ITERS + 3 + _DEPTH - 1) // _DEPTH, ring_body, 0)
    plsc.subcore_barrier()
    _slab_copy(acc, out_hbm.at[c], s)


_agg = pl.kernel(
    _agg_body,
    out_type=jax.ShapeDtypeStruct((NC, N, D), jnp.float32),
    mesh=_MESH,
    scratch_types=(
        [pltpu.VMEM((BLK,), jnp.int32)] * (2 * _DEPTH)
        + [pltpu.VMEM((BLK, D), jnp.float32)] * _RDEPTH
        + [pltpu.SemaphoreType.DMA] * (2 * _DEPTH + 2 * _RDEPTH)
        + [pltpu.VMEM_SHARED((N, D), jnp.float32)]
    ),
)

_MM_ROWS = 2000


def _mm_body(nodes_ref, w_ref, b_ref, degp_ref, out_ref):
    inv = 1.0 / (degp_ref[0] + degp_ref[1] + 1.0)
    h = jnp.dot(nodes_ref[...], w_ref[...], preferred_element_type=jnp.float32)
    h = jnp.maximum(h + b_ref[...], 0.0)
    out_ref[...] = h * inv


def _matmul_scaled(nodes, w, b2d, degp):
    grid = N // _MM_ROWS
    return pl.pallas_call(
        _mm_body,
        grid=(grid,),
        in_specs=[
            pl.BlockSpec((_MM_ROWS, D), lambda i: (i, 0)),
            pl.BlockSpec((D, D), lambda i: (0, 0)),
            pl.BlockSpec((1, D), lambda i: (0, 0)),
            pl.BlockSpec((NC, _MM_ROWS, 1), lambda i: (0, i, 0)),
        ],
        out_specs=pl.BlockSpec((_MM_ROWS, D), lambda i: (i, 0)),
        out_shape=jax.ShapeDtypeStruct((N, D), jnp.float32),
    )(nodes, w, b2d, degp)


def _add_body(p_ref, out_ref):
    out_ref[...] = p_ref[0] + p_ref[1]


def _combine(parts):
    grid = N // _MM_ROWS
    return pl.pallas_call(
        _add_body,
        grid=(grid,),
        in_specs=[pl.BlockSpec((NC, _MM_ROWS, D), lambda i: (0, i, 0))],
        out_specs=pl.BlockSpec((_MM_ROWS, D), lambda i: (i, 0)),
        out_shape=jax.ShapeDtypeStruct((N, D), jnp.float32),
    )(parts)


def kernel(nodes, senders, receivers, W, b):
    s2d = senders.reshape(NB, BLK)
    r2d = receivers.reshape(NB, BLK)
    zerosD = jnp.zeros((N, D), jnp.float32)
    degflat = _hist(s2d)
    degp3 = degflat.reshape(NC, N, 1)
    h_scaled = _matmul_scaled(nodes, W, b.reshape(1, D), degp3)
    parts = _agg(h_scaled, s2d, r2d, zerosD)
    return _combine(parts)


# speedup vs baseline: 16.2217x; 1.0710x over previous
import jax
import jax.numpy as jnp
from jax import lax
from jax.experimental import pallas as pl
from jax.experimental.pallas import tpu as pltpu
from jax.experimental.pallas import tpu_sc as plsc

N = 10000
E = 320000
D = 128
BLK = 128
NB = E // BLK
NC = 2
NS = 16
NW = NC * NS
ROWS_PER_TILE = 624
TAIL0 = NS * ROWS_PER_TILE
TAIL = N - TAIL0
MAX_ITERS = (NB + NW - 1) // NW

_MESH = plsc.VectorSubcoreMesh(
    core_axis_name="c", subcore_axis_name="s", num_cores=NC, num_subcores=NS
)


def _slab_copy(src, dst, s):
    r0 = pl.multiple_of(s * ROWS_PER_TILE, 8)
    pltpu.sync_copy(src.at[pl.ds(r0, ROWS_PER_TILE)],
                    dst.at[pl.ds(r0, ROWS_PER_TILE)])

    @pl.when(s == NS - 1)
    def _():
        pltpu.sync_copy(src.at[pl.ds(TAIL0, TAIL)], dst.at[pl.ds(TAIL0, TAIL)])


def _hist_body(s2d_hbm, out_hbm, sidx0, sidx1, sidx2, sidx3, ones_v, stage_t,
               si0, si1, si2, si3, sc0, sc1, sc2, sc3, dacc):
    c = lax.axis_index("c")
    s = lax.axis_index("s")
    wid = s * NC + c
    sidx = (sidx0, sidx1, sidx2, sidx3)
    sems = (si0, si1, si2, si3)
    sems_c = (sc0, sc1, sc2, sc3)
    r0 = pl.multiple_of(s * ROWS_PER_TILE, 8)

    for i in range(ROWS_PER_TILE // 16):
        stage_t[pl.ds(16 * i, 16)] = jnp.zeros((16,), jnp.float32)

    pltpu.sync_copy(stage_t, dacc.at[pl.ds(r0, ROWS_PER_TILE)])

    @pl.when(s == NS - 1)
    def _():
        pltpu.sync_copy(stage_t.at[pl.ds(0, TAIL)], dacc.at[pl.ds(TAIL0, TAIL)])

    for i in range(BLK // 16):
        ones_v[pl.ds(16 * i, 16)] = jnp.ones((16,), jnp.float32)

    plsc.subcore_barrier()

    for p in range(2):
        @pl.when(wid + NW * p < NB)
        def _():
            pltpu.async_copy(s2d_hbm.at[wid + NW * p], sidx[p], sems[p])

    def ring_body(k0, carry):
        for b in range(4):
            k = 4 * k0 + b
            bw = (b - 2) % 4
            ba = (b + 2) % 4

            @pl.when((k >= 2) & (wid + NW * (k - 2) < NB))
            def _():
                pltpu.make_async_copy(
                    ones_v, dacc.at[sidx[bw]], sems_c[bw]).wait()

            @pl.when(wid + NW * (k + 2) < NB)
            def _():
                pltpu.async_copy(s2d_hbm.at[wid + NW * (k + 2)],
                                 sidx[ba], sems[ba])

            @pl.when(wid + NW * k < NB)
            def _():
                pltpu.make_async_copy(
                    s2d_hbm.at[wid + NW * k], sidx[b], sems[b]).wait()
                pltpu.async_copy(ones_v, dacc.at[sidx[b]], sems_c[b],
                                 add=True)

        return carry

    lax.fori_loop(0, (MAX_ITERS + 2 + 3) // 4, ring_body, 0)
    plsc.subcore_barrier()
    base = pl.multiple_of(c * N + r0, 8)
    pltpu.sync_copy(dacc.at[pl.ds(r0, ROWS_PER_TILE)], stage_t)
    pltpu.sync_copy(stage_t, out_hbm.at[pl.ds(base, ROWS_PER_TILE)])

    @pl.when(s == NS - 1)
    def _():
        pltpu.sync_copy(dacc.at[pl.ds(TAIL0, TAIL)], stage_t.at[pl.ds(0, TAIL)])
        pltpu.sync_copy(stage_t.at[pl.ds(0, TAIL)],
                        out_hbm.at[pl.ds(pl.multiple_of(c * N + TAIL0, 8), TAIL)])


_hist = pl.kernel(
    _hist_body,
    out_type=jax.ShapeDtypeStruct((NC * N,), jnp.float32),
    mesh=_MESH,
    scratch_types=(
        [pltpu.VMEM((BLK,), jnp.int32)] * 4
        + [pltpu.VMEM((BLK,), jnp.float32),
           pltpu.VMEM((ROWS_PER_TILE,), jnp.float32)]
        + [pltpu.SemaphoreType.DMA] * 8
        + [pltpu.VMEM_SHARED((N,), jnp.float32)]
    ),
)


_DEPTH = 6
_RDEPTH = 3


def _agg_body(h_hbm, s2d_hbm, r2d_hbm, out_hbm, *refs):
    sidx = refs[0:_DEPTH]
    ridx = refs[_DEPTH:2 * _DEPTH]
    rows = refs[2 * _DEPTH:2 * _DEPTH + _RDEPTH]
    sems_s = refs[2 * _DEPTH + _RDEPTH:3 * _DEPTH + _RDEPTH]
    sems_r = refs[3 * _DEPTH + _RDEPTH:4 * _DEPTH + _RDEPTH]
    sems_g = refs[4 * _DEPTH + _RDEPTH:4 * _DEPTH + 2 * _RDEPTH]
    sems_c = refs[4 * _DEPTH + 2 * _RDEPTH:4 * _DEPTH + 3 * _RDEPTH]
    acc = refs[4 * _DEPTH + 3 * _RDEPTH]
    c = lax.axis_index("c")
    s = lax.axis_index("s")
    wid = s * NC + c

    _slab_copy(h_hbm, acc, s)
    plsc.subcore_barrier()

    for p in range(3):
        @pl.when(wid + NW * p < NB)
        def _():
            pltpu.async_copy(s2d_hbm.at[wid + NW * p], sidx[p], sems_s[p])
            pltpu.async_copy(r2d_hbm.at[wid + NW * p], ridx[p], sems_r[p])

    def ring_body(k0, carry):
        for b in range(_DEPTH):
            k = _DEPTH * k0 + b
            bw = (b - 3) % _DEPTH
            ba = (b + 3) % _DEPTH
            bg = (b - 2) % _DEPTH
            rw = (b - 3) % _RDEPTH
            rg = (b - 2) % _RDEPTH

            @pl.when((k >= 3) & (wid + NW * (k - 3) < NB))
            def _():
                pltpu.make_async_copy(
                    rows[rw], acc.at[ridx[bw]], sems_c[rw]).wait()

            @pl.when(wid + NW * (k + 3) < NB)
            def _():
                pltpu.async_copy(s2d_hbm.at[wid + NW * (k + 3)],
                                 sidx[ba], sems_s[ba])
                pltpu.async_copy(r2d_hbm.at[wid + NW * (k + 3)],
                                 ridx[ba], sems_r[ba])

            @pl.when(wid + NW * k < NB)
            def _():
                pltpu.make_async_copy(
                    s2d_hbm.at[wid + NW * k], sidx[b], sems_s[b]).wait()
                pltpu.async_copy(h_hbm.at[sidx[b]],
                                 rows[b % _RDEPTH], sems_g[b % _RDEPTH])

            @pl.when((k >= 2) & (wid + NW * (k - 2) < NB))
            def _():
                pltpu.make_async_copy(
                    h_hbm.at[sidx[bg]], rows[rg], sems_g[rg]).wait()
                pltpu.make_async_copy(
                    r2d_hbm.at[wid + NW * (k - 2)], ridx[bg],
                    sems_r[bg]).wait()
                pltpu.async_copy(rows[rg], acc.at[ridx[bg]], sems_c[rg],
                                 add=True)

        return carry

    lax.fori_loop(0, (MAX_ITERS + 3 + _DEPTH - 1) // _DEPTH, ring_body, 0)
    plsc.subcore_barrier()
    _slab_copy(acc, out_hbm.at[c], s)


_agg = pl.kernel(
    _agg_body,
    out_type=jax.ShapeDtypeStruct((NC, N, D), jnp.float32),
    mesh=_MESH,
    scratch_types=(
        [pltpu.VMEM((BLK,), jnp.int32)] * (2 * _DEPTH)
        + [pltpu.VMEM((BLK, D), jnp.float32)] * _RDEPTH
        + [pltpu.SemaphoreType.DMA] * (2 * _DEPTH + 2 * _RDEPTH)
        + [pltpu.VMEM_SHARED((N, D), jnp.float32)]
    ),
)

_MM_ROWS = 2000


def _mm_body(nodes_ref, w_ref, b_ref, degp_ref, out_ref):
    inv = 1.0 / (degp_ref[0] + degp_ref[1] + 1.0)
    h = jnp.dot(nodes_ref[...], w_ref[...], preferred_element_type=jnp.float32)
    h = jnp.maximum(h + b_ref[...], 0.0)
    out_ref[...] = h * inv


def _matmul_scaled(nodes, w, b2d, degp):
    grid = N // _MM_ROWS
    return pl.pallas_call(
        _mm_body,
        grid=(grid,),
        in_specs=[
            pl.BlockSpec((_MM_ROWS, D), lambda i: (i, 0)),
            pl.BlockSpec((D, D), lambda i: (0, 0)),
            pl.BlockSpec((1, D), lambda i: (0, 0)),
            pl.BlockSpec((NC, _MM_ROWS, 1), lambda i: (0, i, 0)),
        ],
        out_specs=pl.BlockSpec((_MM_ROWS, D), lambda i: (i, 0)),
        out_shape=jax.ShapeDtypeStruct((N, D), jnp.float32),
    )(nodes, w, b2d, degp)


def _add_body(p_ref, h_ref, out_ref):
    out_ref[...] = (p_ref[0] - h_ref[...]) + p_ref[1]


def _combine(parts, h_scaled):
    grid = N // _MM_ROWS
    return pl.pallas_call(
        _add_body,
        grid=(grid,),
        in_specs=[
            pl.BlockSpec((NC, _MM_ROWS, D), lambda i: (0, i, 0)),
            pl.BlockSpec((_MM_ROWS, D), lambda i: (i, 0)),
        ],
        out_specs=pl.BlockSpec((_MM_ROWS, D), lambda i: (i, 0)),
        out_shape=jax.ShapeDtypeStruct((N, D), jnp.float32),
    )(parts, h_scaled)


def kernel(nodes, senders, receivers, W, b):
    s2d = senders.reshape(NB, BLK)
    r2d = receivers.reshape(NB, BLK)
    degflat = _hist(s2d)
    degp3 = degflat.reshape(NC, N, 1)
    h_scaled = _matmul_scaled(nodes, W, b.reshape(1, D), degp3)
    parts = _agg(h_scaled, s2d, r2d)
    return _combine(parts, h_scaled)
